# baseline (speedup 1.0000x reference)
import jax
import jax.numpy as jnp
from jax import lax
from jax.experimental import pallas as pl
from jax.experimental.pallas import tpu as pltpu
from jax.experimental.pallas import tpu_sc as plsc

NC, NS = 2, 16
NW = NC * NS
H = 50
B = 16384
D = 64
VOCAB = 1_000_000
BB_PER_W = (B // 128) // NW


def _emb_body(idx_hbm, table_hbm, out_hbm,
              idx_t, rowidx, g0, g1, o0, o1,
              gs0, gs1, os0, os1):
    wid = lax.axis_index("s") * NC + lax.axis_index("c")
    gbuf = (g0, g1)
    obuf = (o0, o1)
    gsem = (gs0, gs1)
    osem = (os0, os1)
    iota16 = lax.iota(jnp.int32, 16)

    def conv_rows(nrows):
        def conv(k, c):
            r = k // 8
            cc = 16 * (k % 8)
            v = idx_t[r, pl.ds(cc, 16)]
            rowidx[r, pl.ds(cc, 16)] = lax.shift_right_logical(v, 1)
            return c
        lax.fori_loop(0, 8 * nrows, conv, 0)

    def unit(hh, h, col0):
        gb = gbuf[hh % 2]
        ob = obuf[hh % 2]
        pltpu.make_async_copy(table_hbm.at[rowidx.at[hh]], gb,
                              gsem[hh % 2]).wait()
        par64 = []
        rowv = []
        for j in range(8):
            iv = idx_t[hh, pl.ds(16 * j, 16)]
            par64.append(lax.shift_left(jnp.bitwise_and(iv, 1), 6))
            rowv.append(iota16 + 16 * j)

        def tbody(dcol, c):
            for j in range(8):
                vals = plsc.load_gather(gb, [rowv[j], par64[j] + dcol])
                ob[dcol, pl.ds(16 * j, 16)] = vals
            return c

        lax.fori_loop(0, D, tbody, 0)
        pltpu.async_copy(
            ob, out_hbm.at[h, pl.ds(0, D), pl.ds(col0, 128)],
            osem[hh % 2])

    def wait_out(k, col0):
        pltpu.make_async_copy(
            obuf[k], out_hbm.at[0, pl.ds(0, D), pl.ds(col0, 128)],
            osem[k]).wait()

    def block(blk, carry):
        bbi = blk // 6
        ho = blk % 6
        col0 = (wid * BB_PER_W + bbi) * 128
        base = 8 * ho

        pltpu.sync_copy(idx_hbm.at[pl.ds(base, 8), pl.ds(col0, 128)], idx_t)
        conv_rows(8)
        pltpu.async_copy(table_hbm.at[rowidx.at[0]], g0, gs0)
        for hh in range(8):
            if hh < 7:
                pltpu.async_copy(table_hbm.at[rowidx.at[hh + 1]],
                                 gbuf[(hh + 1) % 2], gsem[(hh + 1) % 2])
            if hh >= 2:
                wait_out(hh % 2, col0)
            unit(hh, base + hh, col0)
        for k in range(2):
            wait_out(k, col0)
        return carry

    lax.fori_loop(0, 6 * BB_PER_W, block, 0)

    def tail(bbi, carry):
        col0 = (wid * BB_PER_W + bbi) * 128
        pltpu.sync_copy(idx_hbm.at[pl.ds(48, 2), pl.ds(col0, 128)],
                        idx_t.at[pl.ds(0, 2)])
        conv_rows(2)
        pltpu.async_copy(table_hbm.at[rowidx.at[0]], g0, gs0)
        pltpu.async_copy(table_hbm.at[rowidx.at[1]], g1, gs1)
        unit(0, 48, col0)
        unit(1, 49, col0)
        for k in range(2):
            wait_out(k, col0)
        return carry

    lax.fori_loop(0, BB_PER_W, tail, 0)


@jax.jit
def kernel(token_ids, weight):
    idx_t = token_ids.T
    table2 = weight.reshape(VOCAB // 2, 128)
    mesh = plsc.VectorSubcoreMesh(
        core_axis_name="c", subcore_axis_name="s", num_cores=NC, num_subcores=NS
    )
    out = pl.kernel(
        _emb_body,
        out_type=jax.ShapeDtypeStruct((H, D, B), jnp.float32),
        mesh=mesh,
        scratch_types=[
            pltpu.VMEM((8, 128), jnp.int32),
            pltpu.VMEM((8, 128), jnp.int32),
            pltpu.VMEM((128, 128), jnp.float32),
            pltpu.VMEM((128, 128), jnp.float32),
            pltpu.VMEM((D, 128), jnp.float32),
            pltpu.VMEM((D, 128), jnp.float32),
            pltpu.SemaphoreType.DMA,
            pltpu.SemaphoreType.DMA,
            pltpu.SemaphoreType.DMA,
            pltpu.SemaphoreType.DMA,
        ],
        compiler_params=pltpu.CompilerParams(
            use_tc_tiling_on_sc=True, needs_layout_passes=False),
    )(idx_t, table2)
    return out.transpose(2, 0, 1)
